# baseline (speedup 1.0000x reference)
import jax
import jax.numpy as jnp
from jax import lax
from jax.experimental import pallas as pl
from jax.experimental.pallas import tpu as pltpu
from jax.experimental.pallas import tpu_sc as plsc

B = 8192
F = 257
C = 8
N = 4
NBEAMS = 16
HROW_X = F * C
HROW_O = F * N
HROW_XP = 2064
HROW_OP = 1040
ROW_W = N * 2 * F * C
G = 8
HALF = B // 2
NFT = (F + 15) // 16


def _compute_chunk(x_v, w_v, o_v):
    iota = lax.iota(jnp.int32, 16)
    i8 = iota * 8
    i4 = iota * 4

    def f_body(ft, carry):
        f0 = jnp.minimum(ft * 16, F - 16)
        xbase = i8 + f0 * C
        obase = i4 + f0 * N
        for npair in range(2):
            n0 = 2 * npair
            wr = [[w_v[pl.ds((c * N + n0 + k) * F + f0, 16)] for k in range(2)]
                  for c in range(C)]
            wi = [[w_v[pl.ds(((C + c) * N + n0 + k) * F + f0, 16)] for k in range(2)]
                  for c in range(C)]
            for g in range(G):
                xrr = x_v.at[2 * g]
                xri = x_v.at[2 * g + 1]
                ar0 = jnp.zeros((16,), jnp.float32)
                ar1 = jnp.zeros((16,), jnp.float32)
                ai0 = jnp.zeros((16,), jnp.float32)
                ai1 = jnp.zeros((16,), jnp.float32)
                for c in range(C):
                    colr = xbase + c
                    xr = plsc.load_gather(xrr, [colr])
                    xi = plsc.load_gather(xri, [colr])
                    ar0 = ar0 + xr * wr[c][0] + xi * wi[c][0]
                    ar1 = ar1 + xr * wr[c][1] + xi * wi[c][1]
                    ai0 = ai0 + xi * wr[c][0] - xr * wi[c][0]
                    ai1 = ai1 + xi * wr[c][1] - xr * wi[c][1]
                oc = obase + n0
                plsc.store_scatter(o_v.at[2 * g], [oc], ar0)
                plsc.store_scatter(o_v.at[2 * g], [oc + 1], ar1)
                plsc.store_scatter(o_v.at[2 * g + 1], [oc], ai0)
                plsc.store_scatter(o_v.at[2 * g + 1], [oc + 1], ai1)
        return carry

    lax.fori_loop(0, NFT, f_body, 0)


def _sc_body(x_hbm, bid_hbm, w_hbm, out_hbm,
             bid_v, idx_v, w_v, xb0, xb1, ob0, ob1, sg0, sg1, ss0, ss1):
    core = lax.axis_index("c")
    beam = lax.axis_index("s")
    half_base = core * HALF

    pltpu.sync_copy(bid_hbm.at[pl.ds(half_base, HALF)], bid_v)
    pltpu.sync_copy(w_hbm.at[beam], w_v)

    iota = lax.iota(jnp.int32, 16)
    ihalf = iota // 2
    ibit = iota % 2

    def comp_body(i, cursor):
        bid = bid_v[pl.ds(i * 16, 16)]
        mask = bid == beam
        vals = iota + (half_base + i * 16)
        mask_i32 = jnp.where(mask, jnp.int32(1), jnp.int32(0))
        incl = plsc.cumsum(mask_i32)
        pos = cursor + incl - mask_i32
        plsc.store_scatter(idx_v, [pos], vals, mask=mask)
        return cursor + jnp.sum(mask_i32)

    n = lax.fori_loop(0, HALF // 16, comp_body, jnp.int32(0))

    xb = (xb0, xb1)
    ob = (ob0, ob1)
    sg = (sg0, sg1)
    ss = (ss0, ss1)

    def hidx(j):
        ids = plsc.load_gather(idx_v, [ihalf + j * G])
        return ids * 2 + ibit

    @pl.when(n > 0)
    def _():
        pad = plsc.load_gather(idx_v, [jnp.full((16,), n - 1, jnp.int32)])
        idx_v[pl.ds(n, 16)] = pad
        nc2 = (n + 2 * G - 1) // (2 * G)
        last = nc2 * 2 - 1

        pltpu.async_copy(x_hbm.at[hidx(0)], xb[0], sg[0]).wait()

        def pair_body(jj, carry):
            j0 = jj * 2
            j1 = j0 + 1
            h1 = pltpu.async_copy(x_hbm.at[hidx(j1)], xb[1], sg[1])
            _compute_chunk(xb[0], w_v, ob[0])
            s0 = pltpu.async_copy(ob[0], out_hbm.at[hidx(j0)], ss[0])
            h1.wait()
            h2 = pltpu.async_copy(
                x_hbm.at[hidx(jnp.minimum(j0 + 2, last))], xb[0], sg[0])
            _compute_chunk(xb[1], w_v, ob[1])
            s1 = pltpu.async_copy(ob[1], out_hbm.at[hidx(j1)], ss[1])
            s0.wait()
            h2.wait()
            s1.wait()
            return carry

        lax.fori_loop(0, nc2, pair_body, 0)


def _beamform_sc(x_half, bid, w_flat):
    mesh = plsc.VectorSubcoreMesh(
        core_axis_name="c", subcore_axis_name="s",
        num_cores=2, num_subcores=16)
    return pl.kernel(
        _sc_body,
        out_type=jax.ShapeDtypeStruct((2 * B, HROW_OP), jnp.float32),
        mesh=mesh,
        compiler_params=pltpu.CompilerParams(
            needs_layout_passes=False, use_tc_tiling_on_sc=False),
        scratch_types=[
            pltpu.VMEM((HALF,), jnp.int32),
            pltpu.VMEM((HALF + 32,), jnp.int32),
            pltpu.VMEM((ROW_W,), jnp.float32),
            pltpu.VMEM((2 * G, HROW_XP), jnp.float32),
            pltpu.VMEM((2 * G, HROW_XP), jnp.float32),
            pltpu.VMEM((2 * G, HROW_OP), jnp.float32),
            pltpu.VMEM((2 * G, HROW_OP), jnp.float32),
            pltpu.SemaphoreType.DMA,
            pltpu.SemaphoreType.DMA,
            pltpu.SemaphoreType.DMA,
            pltpu.SemaphoreType.DMA,
        ],
    )(x_half, bid, w_flat)


def kernel(input, beam_id, W):
    x_half = jnp.pad(input.reshape(2 * B, HROW_X),
                     ((0, 0), (0, HROW_XP - HROW_X)))
    w_flat = jnp.transpose(W, (0, 2, 4, 1, 3)).reshape(NBEAMS, ROW_W)
    bid = beam_id.astype(jnp.int32)
    out = _beamform_sc(x_half, bid, w_flat)
    return out[:, :HROW_O].reshape(B, 2, F, N)


# speedup vs baseline: 3.0101x; 3.0101x over previous
import jax
import jax.numpy as jnp
from jax import lax
from jax.experimental import pallas as pl
from jax.experimental.pallas import tpu as pltpu
from jax.experimental.pallas import tpu_sc as plsc

B = 8192
F = 257
C = 8
N = 4
NBEAMS = 16
ROW_X = 2 * F * C
ROW_O = 2 * F * N
ROW_W = N * 2 * F * C
G = 16
HALF = B // 2
NFT = (F + 15) // 16


def _compute_chunk(x_v, w_v, o_v):
    iota = lax.iota(jnp.int32, 16)
    i8 = iota * 8
    i4 = iota * 4

    def f_body(ft, carry):
        f0 = jnp.minimum(ft * 16, F - 16)
        for npair in range(2):
            n0 = 2 * npair
            wr = [[w_v[pl.ds((c * N + n0 + k) * F + f0, 16)] for k in range(2)]
                  for c in range(C)]
            wi = [[w_v[pl.ds(((C + c) * N + n0 + k) * F + f0, 16)] for k in range(2)]
                  for c in range(C)]
            @plsc.parallel_loop(0, G, 1, unroll=4)
            def g_body(g):
                xrow = x_v.at[g]
                orow = o_v.at[g]
                ar0 = jnp.zeros((16,), jnp.float32)
                ar1 = jnp.zeros((16,), jnp.float32)
                ai0 = jnp.zeros((16,), jnp.float32)
                ai1 = jnp.zeros((16,), jnp.float32)
                for c in range(C):
                    colr = i8 + (f0 * C + c)
                    xr = plsc.load_gather(xrow, [colr])
                    xi = plsc.load_gather(xrow, [colr + F * C])
                    ar0 = ar0 + xr * wr[c][0] + xi * wi[c][0]
                    ar1 = ar1 + xr * wr[c][1] + xi * wi[c][1]
                    ai0 = ai0 + xi * wr[c][0] - xr * wi[c][0]
                    ai1 = ai1 + xi * wr[c][1] - xr * wi[c][1]
                ob = i4 + f0 * N
                plsc.store_scatter(orow, [ob + n0], ar0)
                plsc.store_scatter(orow, [ob + (n0 + 1)], ar1)
                plsc.store_scatter(orow, [ob + (F * N + n0)], ai0)
                plsc.store_scatter(orow, [ob + (F * N + n0 + 1)], ai1)
        return carry

    lax.fori_loop(0, NFT, f_body, 0)


def _sc_body(x_hbm, bid_hbm, w_hbm, out_hbm,
             bid_v, idx_v, w_v, x_v, o_v, sem_g, sem_s):
    core = lax.axis_index("c")
    beam = lax.axis_index("s")
    half_base = core * HALF

    pltpu.sync_copy(bid_hbm.at[pl.ds(half_base, HALF)], bid_v)
    pltpu.sync_copy(w_hbm.at[beam], w_v)

    iota = lax.iota(jnp.int32, 16)

    def comp_body(i, cursor):
        bid = bid_v[pl.ds(i * 16, 16)]
        mask = bid == beam
        vals = iota + (half_base + i * 16)
        mask_i32 = jnp.where(mask, jnp.int32(1), jnp.int32(0))
        incl = plsc.cumsum(mask_i32)
        pos = cursor + incl - mask_i32
        plsc.store_scatter(idx_v, [pos], vals, mask=mask)
        return cursor + jnp.sum(mask_i32)

    n = lax.fori_loop(0, HALF // 16, comp_body, jnp.int32(0))

    @pl.when(n > 0)
    def _():
        pad = plsc.load_gather(idx_v, [jnp.full((16,), n - 1, jnp.int32)])
        idx_v[pl.ds(n, 16)] = pad
        nchunks = (n + G - 1) // G

        def chunk_body(j, carry):
            ivec = idx_v[pl.ds(j * G, G)]
            pltpu.async_copy(x_hbm.at[ivec], x_v, sem_g).wait()
            _compute_chunk(x_v, w_v, o_v)
            pltpu.async_copy(o_v, out_hbm.at[ivec], sem_s).wait()
            return carry

        lax.fori_loop(0, nchunks, chunk_body, 0)


def _beamform_sc(x_flat, bid, w_flat):
    mesh = plsc.VectorSubcoreMesh(
        core_axis_name="c", subcore_axis_name="s",
        num_cores=2, num_subcores=16)
    return pl.kernel(
        _sc_body,
        out_type=jax.ShapeDtypeStruct((B, ROW_O), jnp.float32),
        mesh=mesh,
        compiler_params=pltpu.CompilerParams(
            needs_layout_passes=False, use_tc_tiling_on_sc=False),
        scratch_types=[
            pltpu.VMEM((HALF,), jnp.int32),
            pltpu.VMEM((HALF + 32,), jnp.int32),
            pltpu.VMEM((ROW_W,), jnp.float32),
            pltpu.VMEM((G, ROW_X), jnp.float32),
            pltpu.VMEM((G, ROW_O), jnp.float32),
            pltpu.SemaphoreType.DMA,
            pltpu.SemaphoreType.DMA,
        ],
    )(x_flat, bid, w_flat)


def kernel(input, beam_id, W):
    x_flat = input.reshape(B, ROW_X)
    w_flat = jnp.transpose(W, (0, 2, 4, 1, 3)).reshape(NBEAMS, ROW_W)
    bid = beam_id.astype(jnp.int32)
    out = _beamform_sc(x_flat, bid, w_flat)
    return out.reshape(B, 2, F, N)


# speedup vs baseline: 4.2335x; 1.4064x over previous
import jax
import jax.numpy as jnp
from jax import lax
from jax.experimental import pallas as pl
from jax.experimental.pallas import tpu as pltpu
from jax.experimental.pallas import tpu_sc as plsc

B = 8192
F = 257
C = 8
N = 4
NBEAMS = 16
ROW_X = 2 * F * C
ROW_O = 2 * F * N
ROW_W = N * 2 * F * C
G = 16
HALF = B // 2
NFT = (F + 15) // 16


def _compute_chunk(x_v, w_v, o_v):
    iota = lax.iota(jnp.int32, 16)
    i8 = iota * 8
    i4 = iota * 4

    def f_body(ft, carry):
        f0 = jnp.minimum(ft * 16, F - 16)
        for npair in range(2):
            n0 = 2 * npair
            for ch in range(2):
                c0 = 4 * ch
                wr = [[w_v[pl.ds((c * N + n0 + k) * F + f0, 16)]
                       for k in range(2)] for c in range(c0, c0 + 4)]
                wi = [[w_v[pl.ds(((C + c) * N + n0 + k) * F + f0, 16)]
                       for k in range(2)] for c in range(c0, c0 + 4)]

                @plsc.parallel_loop(0, G, 1, unroll=1)
                def g_body(g):
                    xrow = x_v.at[g]
                    orow = o_v.at[g]
                    ob = i4 + f0 * N
                    if ch == 0:
                        ar0 = jnp.zeros((16,), jnp.float32)
                        ar1 = jnp.zeros((16,), jnp.float32)
                        ai0 = jnp.zeros((16,), jnp.float32)
                        ai1 = jnp.zeros((16,), jnp.float32)
                    else:
                        ar0 = plsc.load_gather(orow, [ob + n0])
                        ar1 = plsc.load_gather(orow, [ob + (n0 + 1)])
                        ai0 = plsc.load_gather(orow, [ob + (F * N + n0)])
                        ai1 = plsc.load_gather(orow, [ob + (F * N + n0 + 1)])
                    for cc in range(4):
                        colr = i8 + (f0 * C + c0 + cc)
                        xr = plsc.load_gather(xrow, [colr])
                        xi = plsc.load_gather(xrow, [colr + F * C])
                        ar0 = ar0 + xr * wr[cc][0] + xi * wi[cc][0]
                        ar1 = ar1 + xr * wr[cc][1] + xi * wi[cc][1]
                        ai0 = ai0 + xi * wr[cc][0] - xr * wi[cc][0]
                        ai1 = ai1 + xi * wr[cc][1] - xr * wi[cc][1]
                    plsc.store_scatter(orow, [ob + n0], ar0)
                    plsc.store_scatter(orow, [ob + (n0 + 1)], ar1)
                    plsc.store_scatter(orow, [ob + (F * N + n0)], ai0)
                    plsc.store_scatter(orow, [ob + (F * N + n0 + 1)], ai1)
        return carry

    lax.fori_loop(0, NFT, f_body, 0)


def _sc_body(x_hbm, bid_hbm, w_hbm, out_hbm,
             bid_v, idx_v, w_v, x_v, o_v, sem_g, sem_s):
    core = lax.axis_index("c")
    beam = lax.axis_index("s")
    half_base = core * HALF

    pltpu.sync_copy(bid_hbm.at[pl.ds(half_base, HALF)], bid_v)
    pltpu.sync_copy(w_hbm.at[beam], w_v)

    iota = lax.iota(jnp.int32, 16)

    def comp_body(i, cursor):
        bid = bid_v[pl.ds(i * 16, 16)]
        mask = bid == beam
        vals = iota + (half_base + i * 16)
        mask_i32 = jnp.where(mask, jnp.int32(1), jnp.int32(0))
        incl = plsc.cumsum(mask_i32)
        pos = cursor + incl - mask_i32
        plsc.store_scatter(idx_v, [pos], vals, mask=mask)
        return cursor + jnp.sum(mask_i32)

    n = lax.fori_loop(0, HALF // 16, comp_body, jnp.int32(0))

    @pl.when(n > 0)
    def _():
        pad = plsc.load_gather(idx_v, [jnp.full((16,), n - 1, jnp.int32)])
        idx_v[pl.ds(n, 16)] = pad
        nchunks = (n + G - 1) // G

        def chunk_body(j, carry):
            ivec = idx_v[pl.ds(j * G, G)]
            pltpu.async_copy(x_hbm.at[ivec], x_v, sem_g).wait()
            _compute_chunk(x_v, w_v, o_v)
            pltpu.async_copy(o_v, out_hbm.at[ivec], sem_s).wait()
            return carry

        lax.fori_loop(0, nchunks, chunk_body, 0)


def _beamform_sc(x_flat, bid, w_flat):
    mesh = plsc.VectorSubcoreMesh(
        core_axis_name="c", subcore_axis_name="s",
        num_cores=2, num_subcores=16)
    return pl.kernel(
        _sc_body,
        out_type=jax.ShapeDtypeStruct((B, ROW_O), jnp.float32),
        mesh=mesh,
        compiler_params=pltpu.CompilerParams(
            needs_layout_passes=False, use_tc_tiling_on_sc=False),
        scratch_types=[
            pltpu.VMEM((HALF,), jnp.int32),
            pltpu.VMEM((HALF + 32,), jnp.int32),
            pltpu.VMEM((ROW_W,), jnp.float32),
            pltpu.VMEM((G, ROW_X), jnp.float32),
            pltpu.VMEM((G, ROW_O), jnp.float32),
            pltpu.SemaphoreType.DMA,
            pltpu.SemaphoreType.DMA,
        ],
    )(x_flat, bid, w_flat)


def kernel(input, beam_id, W):
    x_flat = input.reshape(B, ROW_X)
    w_flat = jnp.transpose(W, (0, 2, 4, 1, 3)).reshape(NBEAMS, ROW_W)
    bid = beam_id.astype(jnp.int32)
    out = _beamform_sc(x_flat, bid, w_flat)
    return out.reshape(B, 2, F, N)


# speedup vs baseline: 4.2499x; 1.0039x over previous
import jax
import jax.numpy as jnp
from jax import lax
from jax.experimental import pallas as pl
from jax.experimental.pallas import tpu as pltpu
from jax.experimental.pallas import tpu_sc as plsc

B = 8192
F = 257
C = 8
N = 4
NBEAMS = 16
ROW_X = 2 * F * C
ROW_O = 2 * F * N
ROW_W = N * 2 * F * C
G = 16
HALF = B // 2
NFT = (F + 15) // 16


def _compute_chunk(x_v, w_v, o_v):
    iota = lax.iota(jnp.int32, 16)
    i8 = iota * 8
    i4 = iota * 4

    def f_body(ft, carry):
        f0 = jnp.minimum(ft * 16, F - 16)
        for npair in range(2):
            n0 = 2 * npair
            for ch in range(2):
                c0 = 4 * ch
                wr = [[w_v[pl.ds((c * N + n0 + k) * F + f0, 16)]
                       for k in range(2)] for c in range(c0, c0 + 4)]
                wi = [[w_v[pl.ds(((C + c) * N + n0 + k) * F + f0, 16)]
                       for k in range(2)] for c in range(c0, c0 + 4)]

                @plsc.parallel_loop(0, G, 1, unroll=1)
                def g_body(g):
                    xrow = x_v.at[g]
                    orow = o_v.at[g]
                    ob = i4 + f0 * N
                    if ch == 0:
                        ar0 = jnp.zeros((16,), jnp.float32)
                        ar1 = jnp.zeros((16,), jnp.float32)
                        ai0 = jnp.zeros((16,), jnp.float32)
                        ai1 = jnp.zeros((16,), jnp.float32)
                    else:
                        ar0 = plsc.load_gather(orow, [ob + n0])
                        ar1 = plsc.load_gather(orow, [ob + (n0 + 1)])
                        ai0 = plsc.load_gather(orow, [ob + (F * N + n0)])
                        ai1 = plsc.load_gather(orow, [ob + (F * N + n0 + 1)])
                    for cc in range(4):
                        colr = i8 + (f0 * C + c0 + cc)
                        xr = plsc.load_gather(xrow, [colr])
                        xi = plsc.load_gather(xrow, [colr + F * C])
                        ar0 = ar0 + xr * wr[cc][0] + xi * wi[cc][0]
                        ar1 = ar1 + xr * wr[cc][1] + xi * wi[cc][1]
                        ai0 = ai0 + xi * wr[cc][0] - xr * wi[cc][0]
                        ai1 = ai1 + xi * wr[cc][1] - xr * wi[cc][1]
                    plsc.store_scatter(orow, [ob + n0], ar0)
                    plsc.store_scatter(orow, [ob + (n0 + 1)], ar1)
                    plsc.store_scatter(orow, [ob + (F * N + n0)], ai0)
                    plsc.store_scatter(orow, [ob + (F * N + n0 + 1)], ai1)
        return carry

    lax.fori_loop(0, NFT, f_body, 0)


def _sc_body(x_hbm, bid_hbm, w_hbm, out_hbm,
             bid_v, idx_v, w_v, x_v, o_v, sem_g, sem_s):
    core = lax.axis_index("c")
    beam = lax.axis_index("s")
    half_base = core * HALF

    pltpu.sync_copy(bid_hbm.at[pl.ds(half_base, HALF)], bid_v)
    pltpu.sync_copy(w_hbm.at[beam], w_v)

    iota = lax.iota(jnp.int32, 16)

    def comp_body(i, cursor):
        bid = bid_v[pl.ds(i * 16, 16)]
        mask = bid == beam
        vals = iota + (half_base + i * 16)
        mask_i32 = jnp.where(mask, jnp.int32(1), jnp.int32(0))
        incl = plsc.cumsum(mask_i32)
        pos = cursor + incl - mask_i32
        plsc.store_scatter(idx_v, [pos], vals, mask=mask)
        return cursor + jnp.sum(mask_i32)

    n = lax.fori_loop(0, HALF // 16, comp_body, jnp.int32(0))

    @pl.when(n > 0)
    def _():
        pad = plsc.load_gather(idx_v, [jnp.full((16,), n - 1, jnp.int32)])
        idx_v[pl.ds(n, 16)] = pad
        nchunks = (n + G - 1) // G
        last = nchunks - 1

        pltpu.async_copy(x_hbm.at[idx_v[pl.ds(0, G)]], x_v, sem_g).wait()

        def chunk_body(j, carry):
            ivec = idx_v[pl.ds(j * G, G)]
            _compute_chunk(x_v, w_v, o_v)
            s = pltpu.async_copy(o_v, out_hbm.at[ivec], sem_s)
            jn = jnp.minimum(j + 1, last)
            h = pltpu.async_copy(x_hbm.at[idx_v[pl.ds(jn * G, G)]], x_v, sem_g)
            s.wait()
            h.wait()
            return carry

        lax.fori_loop(0, nchunks, chunk_body, 0)


def _beamform_sc(x_flat, bid, w_flat):
    mesh = plsc.VectorSubcoreMesh(
        core_axis_name="c", subcore_axis_name="s",
        num_cores=2, num_subcores=16)
    return pl.kernel(
        _sc_body,
        out_type=jax.ShapeDtypeStruct((B, ROW_O), jnp.float32),
        mesh=mesh,
        compiler_params=pltpu.CompilerParams(
            needs_layout_passes=False, use_tc_tiling_on_sc=False),
        scratch_types=[
            pltpu.VMEM((HALF,), jnp.int32),
            pltpu.VMEM((HALF + 32,), jnp.int32),
            pltpu.VMEM((ROW_W,), jnp.float32),
            pltpu.VMEM((G, ROW_X), jnp.float32),
            pltpu.VMEM((G, ROW_O), jnp.float32),
            pltpu.SemaphoreType.DMA,
            pltpu.SemaphoreType.DMA,
        ],
    )(x_flat, bid, w_flat)


def kernel(input, beam_id, W):
    x_flat = input.reshape(B, ROW_X)
    w_flat = jnp.transpose(W, (0, 2, 4, 1, 3)).reshape(NBEAMS, ROW_W)
    bid = beam_id.astype(jnp.int32)
    out = _beamform_sc(x_flat, bid, w_flat)
    return out.reshape(B, 2, F, N)
